# baseline (speedup 1.0000x reference)
import functools

import jax
import jax.numpy as jnp
from jax import lax
from jax.experimental import pallas as pl
from jax.experimental.pallas import tpu as pltpu
from jax.experimental.pallas import tpu_sc as plsc

_B, _N, _D = 64, 1024, 768
_LANES = 16
_NC, _NS = 2, 16
_NW = _NC * _NS
_ROWS_W = _N // _NW
_CHUNK = _ROWS_W * _D
_NVEC = _CHUNK // _LANES


def _sc_body(x_hbm, pe_hbm, o_hbm, pe_v, buf, _unused_sem):
    c = lax.axis_index("c")
    s = lax.axis_index("s")
    wid = s * _NC + c
    off = wid * _CHUNK
    pltpu.sync_copy(pe_hbm.at[pl.ds(off, _CHUNK)], pe_v)

    def batch_step(b, carry):
        pltpu.sync_copy(x_hbm.at[b, pl.ds(off, _CHUNK)], buf)

        def add_step(i, carry2):
            sl = pl.ds(i * _LANES, _LANES)
            plsc.addupdate(buf.at[sl], pe_v[sl])
            return carry2

        lax.fori_loop(0, _NVEC, add_step, 0, unroll=16)
        pltpu.sync_copy(buf, o_hbm.at[b, pl.ds(off, _CHUNK)])
        return carry

    lax.fori_loop(0, _B, batch_step, 0)


_sc_call = functools.partial(
    pl.kernel,
    out_type=jax.ShapeDtypeStruct((_B, _N * _D), jnp.float32),
    mesh=plsc.VectorSubcoreMesh(core_axis_name="c", subcore_axis_name="s"),
    scratch_types=[
        pltpu.VMEM((_CHUNK,), jnp.float32),
        pltpu.VMEM((_CHUNK,), jnp.float32),
        pltpu.SemaphoreType.DMA,
    ],
)(_sc_body)


def kernel(x, pos_emb):
    b, n, d = x.shape
    out = _sc_call(x.reshape(b, n * d), pos_emb.reshape(n * d))
    return out.reshape(b, n, d)
